# Initial kernel scaffold; baseline (speedup 1.0000x reference)
import jax
import jax.numpy as jnp
from jax.experimental import pallas as pl


def kernel(x_hru, x_channel, x_gw_cell, ei_sw_gw, ei_hydro, ei_sw, ei_gw_sw, ei_self, batch, train_data, Wl, bl, Wr, fc1_w, fc1_b, fc2_w, fc2_b):
    raise NotImplementedError("write your pallas kernel here")


# speedup vs baseline: 1.0972x; 1.0972x over previous
import jax
import jax.numpy as jnp
from jax.experimental import pallas as pl

N = 50000
B = 16
L = 2


def _sage(x_src, x_dst, ei, W_l, b_l, W_r, n_dst):
    src = ei[0]
    dst = ei[1]
    msg = jnp.take(x_src, src, axis=0)
    s = jax.ops.segment_sum(msg, dst, num_segments=n_dst)
    cnt = jax.ops.segment_sum(jnp.ones((src.shape[0], 1), dtype=x_src.dtype), dst, num_segments=n_dst)
    aggr = s / jnp.maximum(cnt, 1.0)
    return aggr @ W_l.T + b_l + x_dst @ W_r.T


def _head_kernel(x_ref, w1_ref, b1_ref, o_ref):
    o_ref[...] = jnp.maximum(jnp.dot(x_ref[...], w1_ref[...], preferred_element_type=jnp.float32) + b1_ref[...], 0.0)


def kernel(x_hru, x_channel, x_gw_cell, ei_sw_gw, ei_hydro, ei_sw, ei_gw_sw, ei_self, batch, train_data, Wl, bl, Wr, fc1_w, fc1_b, fc2_w, fc2_b):
    eis = [ei_sw_gw, ei_hydro, ei_sw, ei_gw_sw, ei_self]
    x_gw = x_gw_cell
    for l in range(L):
        out_gw = _sage(x_hru, x_gw, eis[0], Wl[l, 0], bl[l, 0], Wr[l, 0], N)
        out_ch_hyd = _sage(x_channel, x_channel, eis[1], Wl[l, 1], bl[l, 1], Wr[l, 1], N)
        out_ch_sw = _sage(x_hru, x_channel, eis[2], Wl[l, 2], bl[l, 2], Wr[l, 2], N)
        out_ch_gw = _sage(x_gw, x_channel, eis[3], Wl[l, 3], bl[l, 3], Wr[l, 3], N)
        out_hru = _sage(x_hru, x_hru, eis[4], Wl[l, 4], bl[l, 4], Wr[l, 4], N)
        x_gw = jax.nn.relu(out_gw)
        x_channel = jax.nn.relu((out_ch_hyd + out_ch_sw + out_ch_gw) / 3.0)
        x_hru = jax.nn.relu(out_hru)
    s = jax.ops.segment_sum(x_channel, batch, num_segments=B)
    cnt = jax.ops.segment_sum(jnp.ones((N, 1), dtype=x_channel.dtype), batch, num_segments=B)
    pooled = s / jnp.maximum(cnt, 1.0)
    x = jnp.concatenate([pooled, train_data], axis=1)
    h = pl.pallas_call(
        _head_kernel,
        out_shape=jax.ShapeDtypeStruct((B, fc1_w.shape[0]), jnp.float32),
    )(x, fc1_w.T, jnp.broadcast_to(fc1_b, (B, fc1_b.shape[0])))
    return h @ fc2_w.T + fc2_b


# speedup vs baseline: 7.0996x; 6.4708x over previous
import functools

import jax
import jax.numpy as jnp
from jax import lax
from jax.experimental import pallas as pl
from jax.experimental.pallas import tpu as pltpu
from jax.experimental.pallas import tpu_sc as plsc

N = 50000
E = 800000
D = 64
HALF = 32
B = 16
TD = 16
L = 2
NLAYER_TYPES = 5
SRC_TYPE = (0, 1, 0, 2, 0)
C_EDGE = 2000
N_CHUNK = (E // 16) // C_EDGE
CS = 512
NFULL = (E // 16) // CS
CTAIL = (E // 16) - NFULL * CS
STRIPE_A = 3128
SUBCHUNKS_A = ((0, 512), (512, 512), (1024, 512), (1536, 512), (2048, 512), (2560, 512), (3072, 56))
SUBCHUNKS_B = ((0, 512), (512, 512), (1024, 512), (1536, 512), (2048, 512), (2560, 512), (3072, 8))

_MESH = plsc.VectorSubcoreMesh(core_axis_name="c", subcore_axis_name="s")


def _sc_counts_body(d0, d1, d2, d3, d4, o0, o1, o2, o3, o4, acc, idst, ones, zb, bounce):
    c = lax.axis_index("c")
    s = lax.axis_index("s")
    dsts = [d0, d1, d2, d3, d4]
    outs = [o0, o1, o2, o3, o4]

    def fill(i, _):
        ones[pl.ds(i * 16, 16)] = jnp.ones((16,), jnp.float32)
        zb[pl.ds(i * 16, 16)] = jnp.zeros((16,), jnp.float32)
        return 0

    lax.fori_loop(0, 3200 // 16, fill, 0)

    for e in range(NLAYER_TYPES):
        @pl.when(c == e % 2)
        def _():
            @pl.when(s < 15)
            def _():
                pltpu.sync_copy(zb, acc.at[pl.ds(s * 3200, 3200)])

            @pl.when(s == 15)
            def _():
                pltpu.sync_copy(zb.at[pl.ds(0, 2000)], acc.at[pl.ds(48000, 2000)])

            plsc.subcore_barrier()

            def chunk(j, _):
                off = s * (E // 16) + j * C_EDGE
                pltpu.sync_copy(dsts[e].at[pl.ds(off, C_EDGE)], idst)
                pltpu.sync_copy(ones.at[pl.ds(0, C_EDGE)], acc.at[idst], add=True)
                return 0

            lax.fori_loop(0, N_CHUNK, chunk, 0)
            plsc.subcore_barrier()

            @pl.when(s < 15)
            def _():
                pltpu.sync_copy(acc.at[pl.ds(s * 3200, 3200)], bounce)
                pltpu.sync_copy(bounce, outs[e].at[pl.ds(s * 3200, 3200)])

            @pl.when(s == 15)
            def _():
                pltpu.sync_copy(acc.at[pl.ds(48000, 2000)], bounce.at[pl.ds(0, 2000)])
                pltpu.sync_copy(bounce.at[pl.ds(0, 2000)], outs[e].at[pl.ds(48000, 2000)])

            plsc.subcore_barrier()


def _sc_counts(d0, d1, d2, d3, d4):
    return pl.kernel(
        _sc_counts_body,
        compiler_params=pltpu.CompilerParams(use_tc_tiling_on_sc=False),
        out_type=tuple(jax.ShapeDtypeStruct((N,), jnp.float32) for _ in range(NLAYER_TYPES)),
        mesh=_MESH,
        scratch_types=[
            pltpu.VMEM_SHARED((N,), jnp.float32),
            pltpu.VMEM((C_EDGE,), jnp.int32),
            pltpu.VMEM((3200,), jnp.float32),
            pltpu.VMEM((3200,), jnp.float32),
            pltpu.VMEM((3200,), jnp.float32),
        ],
    )(d0, d1, d2, d3, d4)


def _sc_scatter_body(table, s0, s1, s2, s3, s4, d0, d1, d2, d3, d4, out,
                     acc, isrc, idst, rows, sem):
    c = lax.axis_index("c")
    s = lax.axis_index("s")
    sadjs = [s0, s1, s2, s3, s4]
    dsts = [d0, d1, d2, d3, d4]

    def stripe_io(body_a, body_b):
        @pl.when(s < 15)
        def _():
            for off, sz in SUBCHUNKS_A:
                body_a(s * STRIPE_A + off, sz)

        @pl.when(s == 15)
        def _():
            for off, sz in SUBCHUNKS_B:
                body_b(15 * STRIPE_A + off, sz)

    for e in range(NLAYER_TYPES):
        def fillz(i, _):
            rows[i, pl.ds(0, 16)] = jnp.zeros((16,), jnp.float32)
            rows[i, pl.ds(16, 16)] = jnp.zeros((16,), jnp.float32)
            return 0

        lax.fori_loop(0, CS, fillz, 0)

        def zero(off, sz):
            pltpu.sync_copy(rows.at[pl.ds(0, sz)], acc.at[pl.ds(off, sz)])

        stripe_io(zero, zero)
        plsc.subcore_barrier()

        def chunk(off, n):
            pltpu.sync_copy(sadjs[e].at[pl.ds(c * E + off, n)], isrc.at[pl.ds(0, n)])
            pltpu.sync_copy(dsts[e].at[pl.ds(off, n)], idst.at[pl.ds(0, n)])
            pltpu.async_copy(table.at[isrc.at[pl.ds(0, n)]], rows.at[pl.ds(0, n)], sem).wait()
            pltpu.sync_copy(rows.at[pl.ds(0, n)], acc.at[idst.at[pl.ds(0, n)]], add=True)

        def chunk_loop(j, _):
            chunk(s * (E // 16) + j * CS, CS)
            return 0

        lax.fori_loop(0, NFULL, chunk_loop, 0)
        chunk(s * (E // 16) + NFULL * CS, CTAIL)
        plsc.subcore_barrier()

        slab = (2 * e + c) * N

        def outcopy(off, sz):
            pltpu.sync_copy(acc.at[pl.ds(off, sz)], rows.at[pl.ds(0, sz)])
            pltpu.sync_copy(rows.at[pl.ds(0, sz)], out.at[pl.ds(slab + off, sz)])

        stripe_io(outcopy, outcopy)


def _sc_scatter(table, sadjs, dsts):
    return pl.kernel(
        _sc_scatter_body,
        compiler_params=pltpu.CompilerParams(use_tc_tiling_on_sc=False),
        out_type=jax.ShapeDtypeStruct((2 * NLAYER_TYPES * N, HALF), jnp.float32),
        mesh=_MESH,
        scratch_types=[
            pltpu.VMEM_SHARED((N, HALF), jnp.float32),
            pltpu.VMEM((CS,), jnp.int32),
            pltpu.VMEM((CS,), jnp.int32),
            pltpu.VMEM((CS, HALF), jnp.float32),
            pltpu.SemaphoreType.DMA,
        ],
    )(table, *sadjs, *dsts)


R_POST = 2000


def _tc_post_kernel(s_ref, x_ref, cnt_ref, wl_ref, wr_ref, b_ref, o_ref):
    inv = 1.0 / jnp.maximum(cnt_ref[...], 1.0)

    def zhalf(e, h):
        m = (jnp.dot(s_ref[2 * e], wl_ref[e, 0, h], preferred_element_type=jnp.float32)
             + jnp.dot(s_ref[2 * e + 1], wl_ref[e, 1, h], preferred_element_type=jnp.float32))
        return m * inv[:, e:e + 1]

    def roothalf(t, h):
        return (jnp.dot(x_ref[2 * t], wr_ref[t, 0, h], preferred_element_type=jnp.float32)
                + jnp.dot(x_ref[2 * t + 1], wr_ref[t, 1, h], preferred_element_type=jnp.float32))

    for h in range(2):
        bh = b_ref[...]
        hru = jnp.maximum(zhalf(4, h) + roothalf(0, h) + bh[0 + h:1 + h, :], 0.0)
        ch = jnp.maximum((zhalf(1, h) + zhalf(2, h) + zhalf(3, h)
                          + roothalf(1, h) + bh[2 + h:3 + h, :]) / 3.0, 0.0)
        gw = jnp.maximum(zhalf(0, h) + roothalf(2, h) + bh[4 + h:5 + h, :], 0.0)
        o_ref[0 + h] = hru
        o_ref[2 + h] = ch
        o_ref[4 + h] = gw


def _tc_post(S, X, cntT, wl, wr, bsum):
    grid = (N // R_POST,)
    return pl.pallas_call(
        _tc_post_kernel,
        grid=grid,
        in_specs=[
            pl.BlockSpec((2 * NLAYER_TYPES, R_POST, HALF), lambda i: (0, i, 0)),
            pl.BlockSpec((6, R_POST, HALF), lambda i: (0, i, 0)),
            pl.BlockSpec((R_POST, NLAYER_TYPES), lambda i: (i, 0)),
            pl.BlockSpec((NLAYER_TYPES, 2, 2, HALF, HALF), lambda i: (0, 0, 0, 0, 0)),
            pl.BlockSpec((3, 2, 2, HALF, HALF), lambda i: (0, 0, 0, 0, 0)),
            pl.BlockSpec((6, HALF), lambda i: (0, 0)),
        ],
        out_specs=pl.BlockSpec((6, R_POST, HALF), lambda i: (0, i, 0)),
        out_shape=jax.ShapeDtypeStruct((6, N, HALF), jnp.float32),
    )(S, X, cntT, wl, wr, bsum)


R_POOL = 2000


def _tc_pool_kernel(x0_ref, x1_ref, b_ref, td_ref, w1_ref, b1_ref, w2_ref, o_ref,
                    acc0, acc1, accc):
    i = pl.program_id(0)

    @pl.when(i == 0)
    def _():
        acc0[...] = jnp.zeros_like(acc0)
        acc1[...] = jnp.zeros_like(acc1)
        accc[...] = jnp.zeros_like(accc)

    ids = b_ref[0]
    iota = lax.broadcasted_iota(jnp.int32, (B, R_POOL), 0)
    m = (iota == ids).astype(jnp.float32)
    acc0[...] += jnp.dot(m, x0_ref[0], preferred_element_type=jnp.float32)
    acc1[...] += jnp.dot(m, x1_ref[0], preferred_element_type=jnp.float32)
    accc[...] = accc[...] + jnp.sum(m, axis=1, keepdims=True)

    @pl.when(i == (N // R_POOL) - 1)
    def _():
        cd = jnp.maximum(accc[:, :HALF], 1.0)
        pooled0 = acc0[...] / cd
        pooled1 = acc1[...] / cd
        xcat = jnp.concatenate([pooled0, pooled1, td_ref[...]], axis=1)
        h = jnp.maximum(jnp.dot(xcat, w1_ref[...], preferred_element_type=jnp.float32)
                        + b1_ref[...], 0.0)
        o_ref[...] = h * w2_ref[...]


def _tc_pool_head(X, batch3, train_data, w1T, b1b, w2):
    grid = (N // R_POOL,)
    return pl.pallas_call(
        _tc_pool_kernel,
        grid=grid,
        in_specs=[
            pl.BlockSpec((1, R_POOL, HALF), lambda i: (2, i, 0)),
            pl.BlockSpec((1, R_POOL, HALF), lambda i: (3, i, 0)),
            pl.BlockSpec((1, 1, R_POOL), lambda i: (i, 0, 0)),
            pl.BlockSpec((B, TD), lambda i: (0, 0)),
            pl.BlockSpec((D + TD, 128), lambda i: (0, 0)),
            pl.BlockSpec((B, 128), lambda i: (0, 0)),
            pl.BlockSpec((1, 128), lambda i: (0, 0)),
        ],
        out_specs=pl.BlockSpec((B, 128), lambda i: (0, 0)),
        out_shape=jax.ShapeDtypeStruct((B, 128), jnp.float32),
        scratch_shapes=[
            pltpu.VMEM((B, HALF), jnp.float32),
            pltpu.VMEM((B, HALF), jnp.float32),
            pltpu.VMEM((B, 128), jnp.float32),
        ],
    )(X, X, batch3, train_data, w1T, b1b, w2)


def kernel(x_hru, x_channel, x_gw_cell, ei_sw_gw, ei_hydro, ei_sw, ei_gw_sw,
           ei_self, batch, train_data, Wl, bl, Wr, fc1_w, fc1_b, fc2_w, fc2_b):
    eis = [ei_sw_gw, ei_hydro, ei_sw, ei_gw_sw, ei_self]
    srcs = [ei[0] for ei in eis]
    dsts = [ei[1] for ei in eis]

    def pack(x):
        return jnp.stack([x[:, :HALF], x[:, HALF:]], axis=0)

    X = jnp.concatenate([pack(x_hru), pack(x_channel), pack(x_gw_cell)], axis=0)

    sadjs = [
        jnp.concatenate([srcs[e] + (2 * SRC_TYPE[e] + c) * N for c in range(2)])
        for e in range(NLAYER_TYPES)
    ]

    cnts = _sc_counts(*dsts)
    cntT = jnp.stack(cnts, axis=1)

    wl = jnp.transpose(Wl.reshape(L, 5, 2, HALF, 2, HALF), (0, 1, 4, 2, 5, 3))
    wr_t = jnp.stack([Wr[:, 4], Wr[:, 1] + Wr[:, 2] + Wr[:, 3], Wr[:, 0]], axis=1)
    wr = jnp.transpose(wr_t.reshape(L, 3, 2, HALF, 2, HALF), (0, 1, 4, 2, 5, 3))
    bsum = jnp.stack([bl[:, 4], bl[:, 1] + bl[:, 2] + bl[:, 3], bl[:, 0]],
                     axis=1).reshape(L, 6, HALF)

    for l in range(L):
        S = _sc_scatter(X.reshape(6 * N, HALF), sadjs, dsts)
        S = S.reshape(2 * NLAYER_TYPES, N, HALF)
        X = _tc_post(S, X, cntT, wl[l], wr[l], bsum[l])

    batch3 = batch.reshape(N // R_POOL, 1, R_POOL)
    hw = _tc_pool_head(X, batch3, train_data, fc1_w.T,
                       jnp.broadcast_to(fc1_b, (B, 128)), fc2_w)
    return jnp.sum(hw, axis=1, keepdims=True) + fc2_b


# speedup vs baseline: 7.6843x; 1.0824x over previous
import functools

import jax
import jax.numpy as jnp
from jax import lax
from jax.experimental import pallas as pl
from jax.experimental.pallas import tpu as pltpu
from jax.experimental.pallas import tpu_sc as plsc

N = 50000
E = 800000
D = 64
HALF = 32
B = 16
TD = 16
L = 2
NLAYER_TYPES = 5
SRC_TYPE = (0, 1, 0, 2, 0)
C_EDGE = 2000
N_CHUNK = (E // 16) // C_EDGE
QUART = 16
CS = 1000
NPAIR = (E // 16) // CS // 2
STRIPE_A = 3128
SUBCHUNKS_A = ((0, 1000), (1000, 1000), (2000, 1000), (3000, 128))
SUBCHUNKS_B = ((0, 1000), (1000, 1000), (2000, 1000), (3000, 80))

_MESH = plsc.VectorSubcoreMesh(core_axis_name="c", subcore_axis_name="s")


def _sc_counts_body(d0, d1, d2, d3, d4, o0, o1, o2, o3, o4, acc, idst, ones, zb, bounce):
    c = lax.axis_index("c")
    s = lax.axis_index("s")
    dsts = [d0, d1, d2, d3, d4]
    outs = [o0, o1, o2, o3, o4]

    def fill(i, _):
        ones[pl.ds(i * 16, 16)] = jnp.ones((16,), jnp.float32)
        zb[pl.ds(i * 16, 16)] = jnp.zeros((16,), jnp.float32)
        return 0

    lax.fori_loop(0, 3200 // 16, fill, 0)

    for e in range(NLAYER_TYPES):
        @pl.when(c == e % 2)
        def _():
            @pl.when(s < 15)
            def _():
                pltpu.sync_copy(zb, acc.at[pl.ds(s * 3200, 3200)])

            @pl.when(s == 15)
            def _():
                pltpu.sync_copy(zb.at[pl.ds(0, 2000)], acc.at[pl.ds(48000, 2000)])

            plsc.subcore_barrier()

            def chunk(j, _):
                off = s * (E // 16) + j * C_EDGE
                pltpu.sync_copy(dsts[e].at[pl.ds(off, C_EDGE)], idst)
                pltpu.sync_copy(ones.at[pl.ds(0, C_EDGE)], acc.at[idst], add=True)
                return 0

            lax.fori_loop(0, N_CHUNK, chunk, 0)
            plsc.subcore_barrier()

            @pl.when(s < 15)
            def _():
                pltpu.sync_copy(acc.at[pl.ds(s * 3200, 3200)], bounce)
                pltpu.sync_copy(bounce, outs[e].at[pl.ds(s * 3200, 3200)])

            @pl.when(s == 15)
            def _():
                pltpu.sync_copy(acc.at[pl.ds(48000, 2000)], bounce.at[pl.ds(0, 2000)])
                pltpu.sync_copy(bounce.at[pl.ds(0, 2000)], outs[e].at[pl.ds(48000, 2000)])

            plsc.subcore_barrier()


def _sc_counts(d0, d1, d2, d3, d4):
    return pl.kernel(
        _sc_counts_body,
        compiler_params=pltpu.CompilerParams(use_tc_tiling_on_sc=False),
        out_type=tuple(jax.ShapeDtypeStruct((N,), jnp.float32) for _ in range(NLAYER_TYPES)),
        mesh=_MESH,
        scratch_types=[
            pltpu.VMEM_SHARED((N,), jnp.float32),
            pltpu.VMEM((C_EDGE,), jnp.int32),
            pltpu.VMEM((3200,), jnp.float32),
            pltpu.VMEM((3200,), jnp.float32),
            pltpu.VMEM((3200,), jnp.float32),
        ],
    )(d0, d1, d2, d3, d4)


def _sc_scatter_body(table, s0, s1, s2, s3, s4, d0, d1, d2, d3, d4, out,
                     acc, isrc0, isrc1, idst0, idst1, rows0, rows1, sem0, sem1):
    c = lax.axis_index("c")
    s = lax.axis_index("s")
    sadjs = [s0, s1, s2, s3, s4]
    dsts = [d0, d1, d2, d3, d4]
    ebase = s * (E // 16)

    def stripe_io(body_a, body_b):
        @pl.when(s < 15)
        def _():
            for off, sz in SUBCHUNKS_A:
                body_a(s * STRIPE_A + off, sz)

        @pl.when(s == 15)
        def _():
            for off, sz in SUBCHUNKS_B:
                body_b(15 * STRIPE_A + off, sz)

    for e in range(NLAYER_TYPES):
        for q in range(2):
            qbase = (2 * c + q) * E
            def fillz(i, _):
                rows0[i, pl.ds(0, 16)] = jnp.zeros((16,), jnp.float32)
                return 0

            lax.fori_loop(0, CS, fillz, 0)

            def zero(off, sz):
                pltpu.sync_copy(rows0.at[pl.ds(0, sz)], acc.at[pl.ds(off, sz)])

            stripe_io(zero, zero)
            plsc.subcore_barrier()

            def load_idx(off, ib, db):
                pltpu.sync_copy(sadjs[e].at[pl.ds(qbase + off, CS)], ib)
                pltpu.sync_copy(dsts[e].at[pl.ds(off, CS)], db)

            def gather_start(ib, rb, sem):
                pltpu.async_copy(table.at[ib], rb, sem)

            def gather_wait(ib, rb, sem):
                pltpu.make_async_copy(table.at[ib], rb, sem).wait()

            def scatter(rb, db):
                pltpu.sync_copy(rb, acc.at[db], add=True)

            load_idx(ebase, isrc0, idst0)
            gather_start(isrc0, rows0, sem0)

            def pair(p, _):
                j0 = ebase + 2 * p * CS
                load_idx(j0 + CS, isrc1, idst1)
                gather_start(isrc1, rows1, sem1)
                gather_wait(isrc0, rows0, sem0)
                scatter(rows0, idst0)

                @pl.when(p < NPAIR - 1)
                def _():
                    load_idx(j0 + 2 * CS, isrc0, idst0)
                    gather_start(isrc0, rows0, sem0)

                gather_wait(isrc1, rows1, sem1)
                scatter(rows1, idst1)
                return 0

            lax.fori_loop(0, NPAIR, pair, 0)
            plsc.subcore_barrier()

            slab = (4 * e + 2 * c + q) * N

            def outcopy(off, sz):
                pltpu.sync_copy(acc.at[pl.ds(off, sz)], rows0.at[pl.ds(0, sz)])
                pltpu.sync_copy(rows0.at[pl.ds(0, sz)], out.at[pl.ds(slab + off, sz)])

            stripe_io(outcopy, outcopy)
            plsc.subcore_barrier()


def _sc_scatter(table, sadjs, dsts):
    return pl.kernel(
        _sc_scatter_body,
        compiler_params=pltpu.CompilerParams(use_tc_tiling_on_sc=False),
        out_type=jax.ShapeDtypeStruct((4 * NLAYER_TYPES * N, QUART), jnp.float32),
        mesh=_MESH,
        scratch_types=[
            pltpu.VMEM_SHARED((N, QUART), jnp.float32),
            pltpu.VMEM((CS,), jnp.int32),
            pltpu.VMEM((CS,), jnp.int32),
            pltpu.VMEM((CS,), jnp.int32),
            pltpu.VMEM((CS,), jnp.int32),
            pltpu.VMEM((CS, QUART), jnp.float32),
            pltpu.VMEM((CS, QUART), jnp.float32),
            pltpu.SemaphoreType.DMA,
            pltpu.SemaphoreType.DMA,
        ],
    )(table, *sadjs, *dsts)


R_POST = 1000


def _tc_post_kernel(s_ref, x_ref, cnt_ref, wl_ref, wr_ref, b_ref, o_ref):
    inv = 1.0 / jnp.maximum(cnt_ref[...], 1.0)

    def zhalf(e, h):
        sh0 = jnp.concatenate([s_ref[4 * e], s_ref[4 * e + 1]], axis=1)
        sh1 = jnp.concatenate([s_ref[4 * e + 2], s_ref[4 * e + 3]], axis=1)
        m = (jnp.dot(sh0, wl_ref[e, 0, h], preferred_element_type=jnp.float32)
             + jnp.dot(sh1, wl_ref[e, 1, h], preferred_element_type=jnp.float32))
        return m * inv[:, e:e + 1]

    def roothalf(t, h):
        return (jnp.dot(x_ref[2 * t], wr_ref[t, 0, h], preferred_element_type=jnp.float32)
                + jnp.dot(x_ref[2 * t + 1], wr_ref[t, 1, h], preferred_element_type=jnp.float32))

    for h in range(2):
        bh = b_ref[...]
        hru = jnp.maximum(zhalf(4, h) + roothalf(0, h) + bh[0 + h:1 + h, :], 0.0)
        ch = jnp.maximum((zhalf(1, h) + zhalf(2, h) + zhalf(3, h)
                          + roothalf(1, h) + bh[2 + h:3 + h, :]) / 3.0, 0.0)
        gw = jnp.maximum(zhalf(0, h) + roothalf(2, h) + bh[4 + h:5 + h, :], 0.0)
        o_ref[0 + h] = hru
        o_ref[2 + h] = ch
        o_ref[4 + h] = gw


def _tc_post(S, X, cntT, wl, wr, bsum):
    grid = (N // R_POST,)
    return pl.pallas_call(
        _tc_post_kernel,
        grid=grid,
        in_specs=[
            pl.BlockSpec((4 * NLAYER_TYPES, R_POST, QUART), lambda i: (0, i, 0)),
            pl.BlockSpec((6, R_POST, HALF), lambda i: (0, i, 0)),
            pl.BlockSpec((R_POST, NLAYER_TYPES), lambda i: (i, 0)),
            pl.BlockSpec((NLAYER_TYPES, 2, 2, HALF, HALF), lambda i: (0, 0, 0, 0, 0)),
            pl.BlockSpec((3, 2, 2, HALF, HALF), lambda i: (0, 0, 0, 0, 0)),
            pl.BlockSpec((6, HALF), lambda i: (0, 0)),
        ],
        out_specs=pl.BlockSpec((6, R_POST, HALF), lambda i: (0, i, 0)),
        out_shape=jax.ShapeDtypeStruct((6, N, HALF), jnp.float32),
    )(S, X, cntT, wl, wr, bsum)


R_POOL = 2000


def _tc_pool_kernel(x0_ref, x1_ref, b_ref, td_ref, w1_ref, b1_ref, w2_ref, o_ref,
                    acc0, acc1, accc):
    i = pl.program_id(0)

    @pl.when(i == 0)
    def _():
        acc0[...] = jnp.zeros_like(acc0)
        acc1[...] = jnp.zeros_like(acc1)
        accc[...] = jnp.zeros_like(accc)

    ids = b_ref[0]
    iota = lax.broadcasted_iota(jnp.int32, (B, R_POOL), 0)
    m = (iota == ids).astype(jnp.float32)
    acc0[...] += jnp.dot(m, x0_ref[0], preferred_element_type=jnp.float32)
    acc1[...] += jnp.dot(m, x1_ref[0], preferred_element_type=jnp.float32)
    accc[...] = accc[...] + jnp.sum(m, axis=1, keepdims=True)

    @pl.when(i == (N // R_POOL) - 1)
    def _():
        cd = jnp.maximum(accc[:, :HALF], 1.0)
        pooled0 = acc0[...] / cd
        pooled1 = acc1[...] / cd
        xcat = jnp.concatenate([pooled0, pooled1, td_ref[...]], axis=1)
        h = jnp.maximum(jnp.dot(xcat, w1_ref[...], preferred_element_type=jnp.float32)
                        + b1_ref[...], 0.0)
        o_ref[...] = h * w2_ref[...]


def _tc_pool_head(X, batch3, train_data, w1T, b1b, w2):
    grid = (N // R_POOL,)
    return pl.pallas_call(
        _tc_pool_kernel,
        grid=grid,
        in_specs=[
            pl.BlockSpec((1, R_POOL, HALF), lambda i: (2, i, 0)),
            pl.BlockSpec((1, R_POOL, HALF), lambda i: (3, i, 0)),
            pl.BlockSpec((1, 1, R_POOL), lambda i: (i, 0, 0)),
            pl.BlockSpec((B, TD), lambda i: (0, 0)),
            pl.BlockSpec((D + TD, 128), lambda i: (0, 0)),
            pl.BlockSpec((B, 128), lambda i: (0, 0)),
            pl.BlockSpec((1, 128), lambda i: (0, 0)),
        ],
        out_specs=pl.BlockSpec((B, 128), lambda i: (0, 0)),
        out_shape=jax.ShapeDtypeStruct((B, 128), jnp.float32),
        scratch_shapes=[
            pltpu.VMEM((B, HALF), jnp.float32),
            pltpu.VMEM((B, HALF), jnp.float32),
            pltpu.VMEM((B, 128), jnp.float32),
        ],
    )(X, X, batch3, train_data, w1T, b1b, w2)


def kernel(x_hru, x_channel, x_gw_cell, ei_sw_gw, ei_hydro, ei_sw, ei_gw_sw,
           ei_self, batch, train_data, Wl, bl, Wr, fc1_w, fc1_b, fc2_w, fc2_b):
    eis = [ei_sw_gw, ei_hydro, ei_sw, ei_gw_sw, ei_self]
    srcs = [ei[0] for ei in eis]
    dsts = [ei[1] for ei in eis]

    def pack(x):
        return jnp.stack([x[:, :HALF], x[:, HALF:]], axis=0)

    X = jnp.concatenate([pack(x_hru), pack(x_channel), pack(x_gw_cell)], axis=0)

    sadjs = [
        jnp.concatenate([2 * srcs[e] + (2 * (2 * SRC_TYPE[e] + qc // 2) * N + qc % 2)
                         for qc in range(4)])
        for e in range(NLAYER_TYPES)
    ]

    cnts = _sc_counts(*dsts)
    cntT = jnp.stack(cnts, axis=1)

    wl = jnp.transpose(Wl.reshape(L, 5, 2, HALF, 2, HALF), (0, 1, 4, 2, 5, 3))
    wr_t = jnp.stack([Wr[:, 4], Wr[:, 1] + Wr[:, 2] + Wr[:, 3], Wr[:, 0]], axis=1)
    wr = jnp.transpose(wr_t.reshape(L, 3, 2, HALF, 2, HALF), (0, 1, 4, 2, 5, 3))
    bsum = jnp.stack([bl[:, 4], bl[:, 1] + bl[:, 2] + bl[:, 3], bl[:, 0]],
                     axis=1).reshape(L, 6, HALF)

    for l in range(L):
        S = _sc_scatter(X.reshape(6 * N * 2, QUART), sadjs, dsts)
        S = S.reshape(4 * NLAYER_TYPES, N, QUART)
        X = _tc_post(S, X, cntT, wl[l], wr[l], bsum[l])

    batch3 = batch.reshape(N // R_POOL, 1, R_POOL)
    hw = _tc_pool_head(X, batch3, train_data, fc1_w.T,
                       jnp.broadcast_to(fc1_b, (B, 128)), fc2_w)
    return jnp.sum(hw, axis=1, keepdims=True) + fc2_b


# speedup vs baseline: 8.2366x; 1.0719x over previous
import functools

import jax
import jax.numpy as jnp
from jax import lax
from jax.experimental import pallas as pl
from jax.experimental.pallas import tpu as pltpu
from jax.experimental.pallas import tpu_sc as plsc

N = 50000
E = 800000
D = 64
HALF = 32
B = 16
TD = 16
L = 2
NLAYER_TYPES = 5
SRC_TYPE = (0, 1, 0, 2, 0)
C_EDGE = 2000
N_CHUNK = (E // 16) // C_EDGE
QUART = 16
CS = 1000
NPAIR = (E // 16) // CS // 2
STRIPE_A = 3128
SUBCHUNKS_A = ((0, 1000), (1000, 1000), (2000, 1000), (3000, 128))
SUBCHUNKS_B = ((0, 1000), (1000, 1000), (2000, 1000), (3000, 80))

_MESH = plsc.VectorSubcoreMesh(core_axis_name="c", subcore_axis_name="s")


def _sc_counts_body(d0, d1, d2, d3, d4, o0, o1, o2, o3, o4, acc, idst, ones, zb, bounce):
    c = lax.axis_index("c")
    s = lax.axis_index("s")
    dsts = [d0, d1, d2, d3, d4]
    outs = [o0, o1, o2, o3, o4]

    def fill(i, _):
        ones[pl.ds(i * 16, 16)] = jnp.ones((16,), jnp.float32)
        zb[pl.ds(i * 16, 16)] = jnp.zeros((16,), jnp.float32)
        return 0

    lax.fori_loop(0, 3200 // 16, fill, 0)

    for e in range(NLAYER_TYPES):
        @pl.when(c == e % 2)
        def _():
            @pl.when(s < 15)
            def _():
                pltpu.sync_copy(zb, acc.at[pl.ds(s * 3200, 3200)])

            @pl.when(s == 15)
            def _():
                pltpu.sync_copy(zb.at[pl.ds(0, 2000)], acc.at[pl.ds(48000, 2000)])

            plsc.subcore_barrier()

            def chunk(j, _):
                off = s * (E // 16) + j * C_EDGE
                pltpu.sync_copy(dsts[e].at[pl.ds(off, C_EDGE)], idst)
                pltpu.sync_copy(ones.at[pl.ds(0, C_EDGE)], acc.at[idst], add=True)
                return 0

            lax.fori_loop(0, N_CHUNK, chunk, 0)
            plsc.subcore_barrier()

            @pl.when(s < 15)
            def _():
                pltpu.sync_copy(acc.at[pl.ds(s * 3200, 3200)], bounce)
                pltpu.sync_copy(bounce, outs[e].at[pl.ds(s * 3200, 3200)])

            @pl.when(s == 15)
            def _():
                pltpu.sync_copy(acc.at[pl.ds(48000, 2000)], bounce.at[pl.ds(0, 2000)])
                pltpu.sync_copy(bounce.at[pl.ds(0, 2000)], outs[e].at[pl.ds(48000, 2000)])

            plsc.subcore_barrier()


def _sc_counts(d0, d1, d2, d3, d4):
    return pl.kernel(
        _sc_counts_body,
        compiler_params=pltpu.CompilerParams(use_tc_tiling_on_sc=False),
        out_type=tuple(jax.ShapeDtypeStruct((N,), jnp.float32) for _ in range(NLAYER_TYPES)),
        mesh=_MESH,
        scratch_types=[
            pltpu.VMEM_SHARED((N,), jnp.float32),
            pltpu.VMEM((C_EDGE,), jnp.int32),
            pltpu.VMEM((3200,), jnp.float32),
            pltpu.VMEM((3200,), jnp.float32),
            pltpu.VMEM((3200,), jnp.float32),
        ],
    )(d0, d1, d2, d3, d4)


TSEL = (0, 0, 0, 1, 0)
GBASE = (0, 4, 0, 0, 0)
SMAP = ((1, 4), (0, 0), (0, 4), (1, 0), (2, 0))


def _sc_scatter_body(tA, tB, s0, s1, s2, s3, s4, d0, d1, d2, d3, d4,
                     oA, oB, oC,
                     acc, isrc0, isrc1, idst0, idst1, rows0, rows1, sem0, sem1):
    c = lax.axis_index("c")
    s = lax.axis_index("s")
    tables = [tA, tB]
    sadjs = [s0, s1, s2, s3, s4]
    dsts = [d0, d1, d2, d3, d4]
    outs = [oA, oB, oC]
    ebase = s * (E // 16)

    def stripe_io(body):
        @pl.when(s < 15)
        def _():
            for off, sz in SUBCHUNKS_A:
                body(s * STRIPE_A + off, sz)

        @pl.when(s == 15)
        def _():
            for off, sz in SUBCHUNKS_B:
                body(15 * STRIPE_A + off, sz)

    for e in range(NLAYER_TYPES):
        table = tables[TSEL[e]]
        oi, goff = SMAP[e]
        for q in range(2):
            qbase = (2 * c + q) * E
            g = goff + 2 * c + q
            def fillz(i, _):
                rows0[i, pl.ds(0, 16)] = jnp.zeros((16,), jnp.float32)
                return 0

            lax.fori_loop(0, CS, fillz, 0)

            def zero(off, sz):
                pltpu.sync_copy(rows0.at[pl.ds(0, sz)], acc.at[pl.ds(off, sz)])

            stripe_io(zero)
            plsc.subcore_barrier()

            def load_idx(off, ib, db):
                pltpu.sync_copy(sadjs[e].at[pl.ds(qbase + off, CS)], ib)
                pltpu.sync_copy(dsts[e].at[pl.ds(off, CS)], db)

            def gather_start(ib, rb, sem):
                pltpu.async_copy(table.at[ib], rb, sem)

            def gather_wait(ib, rb, sem):
                pltpu.make_async_copy(table.at[ib], rb, sem).wait()

            def scatter(rb, db):
                pltpu.sync_copy(rb, acc.at[db], add=True)

            load_idx(ebase, isrc0, idst0)
            gather_start(isrc0, rows0, sem0)

            def pair(p, _):
                j0 = ebase + 2 * p * CS
                load_idx(j0 + CS, isrc1, idst1)
                gather_start(isrc1, rows1, sem1)
                gather_wait(isrc0, rows0, sem0)
                scatter(rows0, idst0)

                @pl.when(p < NPAIR - 1)
                def _():
                    load_idx(j0 + 2 * CS, isrc0, idst0)
                    gather_start(isrc0, rows0, sem0)

                gather_wait(isrc1, rows1, sem1)
                scatter(rows1, idst1)
                return 0

            lax.fori_loop(0, NPAIR, pair, 0)
            plsc.subcore_barrier()

            def outcopy(off, sz):
                pltpu.sync_copy(acc.at[pl.ds(off, sz)], rows0.at[pl.ds(0, sz)])
                pltpu.sync_copy(rows0.at[pl.ds(0, sz)],
                                outs[oi].at[pl.ds(off, sz), g])

            stripe_io(outcopy)
            plsc.subcore_barrier()


def _sc_scatter(tA, tB, sadjs, dsts):
    return pl.kernel(
        _sc_scatter_body,
        compiler_params=pltpu.CompilerParams(use_tc_tiling_on_sc=False),
        out_type=tuple(jax.ShapeDtypeStruct((N, 8, QUART), jnp.float32) for _ in range(3)),
        mesh=_MESH,
        scratch_types=[
            pltpu.VMEM_SHARED((N, QUART), jnp.float32),
            pltpu.VMEM((CS,), jnp.int32),
            pltpu.VMEM((CS,), jnp.int32),
            pltpu.VMEM((CS,), jnp.int32),
            pltpu.VMEM((CS,), jnp.int32),
            pltpu.VMEM((CS, QUART), jnp.float32),
            pltpu.VMEM((CS, QUART), jnp.float32),
            pltpu.SemaphoreType.DMA,
            pltpu.SemaphoreType.DMA,
        ],
    )(tA, tB, *sadjs, *dsts)


R_POST = 2000


def _tc_post_kernel(sa_ref, sb_ref, sc_ref, xa_ref, xb_ref, cnt_ref, w_ref,
                    b_ref, oa_ref, ob_ref):
    inv = 1.0 / jnp.maximum(cnt_ref[...], 1.0)
    lane = lax.broadcasted_iota(jnp.int32, (R_POST, 128), 1)
    low = lane < 64

    def scaled(ref, el, eh):
        pat = jnp.where(low, inv[:, el:el + 1], inv[:, eh:eh + 1])
        return ref[...] * pat

    sa = scaled(sa_ref, 1, 2)
    sb = scaled(sb_ref, 3, 0)
    sc = jnp.where(low, sc_ref[...] * inv[:, 4:5], 0.0)

    def mm(x, k):
        return jnp.dot(x, w_ref[k], preferred_element_type=jnp.float32)

    z_ch = mm(sa, 0) + mm(sb, 1)
    z_gw = mm(sb, 2)
    z_hru = mm(sc, 3)
    r_gw = mm(xb_ref[...], 4)
    r_ch = mm(xa_ref[...], 5)
    r_hru = mm(xa_ref[...], 6)
    hru = jnp.maximum(z_hru + r_hru + b_ref[0:1, :], 0.0)
    ch = jnp.maximum((z_ch + r_ch + b_ref[1:2, :]) / 3.0, 0.0)
    gw = jnp.maximum(z_gw + r_gw + b_ref[2:3, :], 0.0)
    oa_ref[...] = jnp.concatenate([hru, ch], axis=1)
    ob_ref[...] = jnp.concatenate([gw, hru], axis=1)


def _tc_post(SA, SB, SC2, XA, XB, cntT, w, bsum):
    grid = (N // R_POST,)
    blk = pl.BlockSpec((R_POST, 128), lambda i: (i, 0))
    return pl.pallas_call(
        _tc_post_kernel,
        grid=grid,
        in_specs=[
            blk, blk, blk, blk, blk,
            pl.BlockSpec((R_POST, NLAYER_TYPES), lambda i: (i, 0)),
            pl.BlockSpec((7, 128, D), lambda i: (0, 0, 0)),
            pl.BlockSpec((3, D), lambda i: (0, 0)),
        ],
        out_specs=[blk, blk],
        out_shape=[jax.ShapeDtypeStruct((N, 128), jnp.float32),
                   jax.ShapeDtypeStruct((N, 128), jnp.float32)],
    )(SA, SB, SC2, XA, XB, cntT, w, bsum)


R_POOL = 2000


def _tc_pool_kernel(xa_ref, b_ref, w1p_ref, w1t_ref, td_ref, b1_ref, w2_ref,
                    o_ref, accp, accc):
    i = pl.program_id(0)

    @pl.when(i == 0)
    def _():
        accp[...] = jnp.zeros_like(accp)
        accc[...] = jnp.zeros_like(accc)

    ids = b_ref[0]
    iota = lax.broadcasted_iota(jnp.int32, (B, R_POOL), 0)
    m = (iota == ids).astype(jnp.float32)
    accp[...] += jnp.dot(m, xa_ref[...], preferred_element_type=jnp.float32)
    accc[...] = accc[...] + jnp.sum(m, axis=1, keepdims=True)

    @pl.when(i == (N // R_POOL) - 1)
    def _():
        pooled = accp[...] / jnp.maximum(accc[...], 1.0)
        h = jnp.maximum(jnp.dot(pooled, w1p_ref[...], preferred_element_type=jnp.float32)
                        + jnp.dot(td_ref[...], w1t_ref[...], preferred_element_type=jnp.float32)
                        + b1_ref[...], 0.0)
        o_ref[...] = h * w2_ref[...]


def _tc_pool_head(XA, batch3, w1p, w1t, train_data, b1b, w2):
    grid = (N // R_POOL,)
    return pl.pallas_call(
        _tc_pool_kernel,
        grid=grid,
        in_specs=[
            pl.BlockSpec((R_POOL, 128), lambda i: (i, 0)),
            pl.BlockSpec((1, 1, R_POOL), lambda i: (i, 0, 0)),
            pl.BlockSpec((128, 128), lambda i: (0, 0)),
            pl.BlockSpec((TD, 128), lambda i: (0, 0)),
            pl.BlockSpec((B, TD), lambda i: (0, 0)),
            pl.BlockSpec((B, 128), lambda i: (0, 0)),
            pl.BlockSpec((1, 128), lambda i: (0, 0)),
        ],
        out_specs=pl.BlockSpec((B, 128), lambda i: (0, 0)),
        out_shape=jax.ShapeDtypeStruct((B, 128), jnp.float32),
        scratch_shapes=[
            pltpu.VMEM((B, 128), jnp.float32),
            pltpu.VMEM((B, 128), jnp.float32),
        ],
    )(XA, batch3, w1p, w1t, train_data, b1b, w2)


def kernel(x_hru, x_channel, x_gw_cell, ei_sw_gw, ei_hydro, ei_sw, ei_gw_sw,
           ei_self, batch, train_data, Wl, bl, Wr, fc1_w, fc1_b, fc2_w, fc2_b):
    eis = [ei_sw_gw, ei_hydro, ei_sw, ei_gw_sw, ei_self]
    srcs = [ei[0] for ei in eis]
    dsts = [ei[1] for ei in eis]

    XA = jnp.concatenate([x_hru, x_channel], axis=1)
    XB = jnp.concatenate([x_gw_cell, x_hru], axis=1)

    sadjs = [
        jnp.concatenate([8 * srcs[e] + (GBASE[e] + qc) for qc in range(4)])
        for e in range(NLAYER_TYPES)
    ]

    cnts = _sc_counts(*dsts)
    cntT = jnp.stack(cnts, axis=1)

    z64 = jnp.zeros((64, 64), jnp.float32)

    def wstack(l):
        wr_ch = (Wr[l, 1] + Wr[l, 2] + Wr[l, 3]).T
        return jnp.stack([
            jnp.concatenate([Wl[l, 1].T, Wl[l, 2].T], axis=0),
            jnp.concatenate([Wl[l, 3].T, z64], axis=0),
            jnp.concatenate([z64, Wl[l, 0].T], axis=0),
            jnp.concatenate([Wl[l, 4].T, z64], axis=0),
            jnp.concatenate([Wr[l, 0].T, z64], axis=0),
            jnp.concatenate([z64, wr_ch], axis=0),
            jnp.concatenate([Wr[l, 4].T, z64], axis=0),
        ])
    w = jnp.stack([wstack(l) for l in range(L)])
    bsum = jnp.stack([bl[:, 4], bl[:, 1] + bl[:, 2] + bl[:, 3], bl[:, 0]],
                     axis=1)

    for l in range(L):
        SA, SB, SC2 = _sc_scatter(XA.reshape(8 * N, QUART), XB.reshape(8 * N, QUART),
                                  sadjs, dsts)
        XA, XB = _tc_post(SA.reshape(N, 128), SB.reshape(N, 128),
                          SC2.reshape(N, 128), XA, XB, cntT, w[l], bsum[l])

    batch3 = batch.reshape(N // R_POOL, 1, R_POOL)
    w1p = jnp.concatenate([jnp.zeros((64, 128), jnp.float32), fc1_w.T[:D]], axis=0)
    w1t = fc1_w.T[D:]
    hw = _tc_pool_head(XA, batch3, w1p, w1t, train_data,
                       jnp.broadcast_to(fc1_b, (B, 128)), fc2_w)
    return jnp.sum(hw, axis=1, keepdims=True) + fc2_b


# speedup vs baseline: 11.0984x; 1.3474x over previous
import functools

import jax
import jax.numpy as jnp
from jax import lax
from jax.experimental import pallas as pl
from jax.experimental.pallas import tpu as pltpu
from jax.experimental.pallas import tpu_sc as plsc

N = 50000
E = 800000
D = 64
HALF = 32
B = 16
TD = 16
L = 2
NLAYER_TYPES = 5
SRC_TYPE = (0, 1, 0, 2, 0)
C_EDGE = 2000
N_CHUNK = (E // 16) // C_EDGE
QUART = 16
CS = 1000
NPAIR = (E // 16) // CS // 2
STRIPE_A = 3128
SUBCHUNKS_A = ((0, 1000), (1000, 1000), (2000, 1000), (3000, 128))
SUBCHUNKS_B = ((0, 1000), (1000, 1000), (2000, 1000), (3000, 80))

_MESH = plsc.VectorSubcoreMesh(core_axis_name="c", subcore_axis_name="s")


def _sc_counts_body(d0, d1, d2, d3, d4, o0, o1, o2, o3, o4, acc, idst, ones, zb, bounce):
    c = lax.axis_index("c")
    s = lax.axis_index("s")
    dsts = [d0, d1, d2, d3, d4]
    outs = [o0, o1, o2, o3, o4]

    def fill(i, _):
        ones[pl.ds(i * 16, 16)] = jnp.ones((16,), jnp.float32)
        zb[pl.ds(i * 16, 16)] = jnp.zeros((16,), jnp.float32)
        return 0

    lax.fori_loop(0, 3200 // 16, fill, 0)

    for e in range(NLAYER_TYPES):
        @pl.when(c == e % 2)
        def _():
            @pl.when(s < 15)
            def _():
                pltpu.sync_copy(zb, acc.at[pl.ds(s * 3200, 3200)])

            @pl.when(s == 15)
            def _():
                pltpu.sync_copy(zb.at[pl.ds(0, 2000)], acc.at[pl.ds(48000, 2000)])

            plsc.subcore_barrier()

            def chunk(j, _):
                off = s * (E // 16) + j * C_EDGE
                pltpu.sync_copy(dsts[e].at[pl.ds(off, C_EDGE)], idst)
                pltpu.sync_copy(ones.at[pl.ds(0, C_EDGE)], acc.at[idst], add=True)
                return 0

            lax.fori_loop(0, N_CHUNK, chunk, 0)
            plsc.subcore_barrier()

            @pl.when(s < 15)
            def _():
                pltpu.sync_copy(acc.at[pl.ds(s * 3200, 3200)], bounce)
                pltpu.sync_copy(bounce, outs[e].at[pl.ds(s * 3200, 3200)])

            @pl.when(s == 15)
            def _():
                pltpu.sync_copy(acc.at[pl.ds(48000, 2000)], bounce.at[pl.ds(0, 2000)])
                pltpu.sync_copy(bounce.at[pl.ds(0, 2000)], outs[e].at[pl.ds(48000, 2000)])

            plsc.subcore_barrier()


def _sc_counts(d0, d1, d2, d3, d4):
    return pl.kernel(
        _sc_counts_body,
        compiler_params=pltpu.CompilerParams(use_tc_tiling_on_sc=False),
        out_type=tuple(jax.ShapeDtypeStruct((N,), jnp.float32) for _ in range(NLAYER_TYPES)),
        mesh=_MESH,
        scratch_types=[
            pltpu.VMEM_SHARED((N,), jnp.float32),
            pltpu.VMEM((C_EDGE,), jnp.int32),
            pltpu.VMEM((3200,), jnp.float32),
            pltpu.VMEM((3200,), jnp.float32),
            pltpu.VMEM((3200,), jnp.float32),
        ],
    )(d0, d1, d2, d3, d4)


TSEL = (0, 0, 0, 1, 0)
GBASE = (0, 4, 0, 0, 0)
SMAP = ((1, 4), (0, 0), (0, 4), (1, 0), (2, 0))


def _sc_scatter_body(tA, tB, s0, s1, s2, s3, s4, d0, d1, d2, d3, d4,
                     oA, oB, oC,
                     acc, isrc0, isrc1, idst0, idst1, rows0, rows1, sem0, sem1):
    c = lax.axis_index("c")
    s = lax.axis_index("s")
    tables = [tA, tB]
    sadjs = [s0, s1, s2, s3, s4]
    dsts = [d0, d1, d2, d3, d4]
    outs = [oA, oB, oC]
    ebase = s * (E // 16)

    def stripe_io(body):
        @pl.when(s < 15)
        def _():
            for off, sz in SUBCHUNKS_A:
                body(s * STRIPE_A + off, sz)

        @pl.when(s == 15)
        def _():
            for off, sz in SUBCHUNKS_B:
                body(15 * STRIPE_A + off, sz)

    for e in range(NLAYER_TYPES):
        table = tables[TSEL[e]]
        oi, goff = SMAP[e]
        for q in range(2):
            qbase = (2 * c + q) * E
            g = goff + 2 * c + q
            def fillz(i, _):
                rows0[i, pl.ds(0, 16)] = jnp.zeros((16,), jnp.float32)
                return 0

            lax.fori_loop(0, CS, fillz, 0)

            def zero(off, sz):
                pltpu.sync_copy(rows0.at[pl.ds(0, sz)], acc.at[pl.ds(off, sz)])

            stripe_io(zero)
            plsc.subcore_barrier()

            def load_idx(off, ib, db):
                pltpu.sync_copy(sadjs[e].at[pl.ds(qbase + off, CS)], ib)
                pltpu.sync_copy(dsts[e].at[pl.ds(off, CS)], db)

            def gather_start(ib, rb, sem):
                pltpu.async_copy(table.at[ib], rb, sem)

            def gather_wait(ib, rb, sem):
                pltpu.make_async_copy(table.at[ib], rb, sem).wait()

            def scatter(rb, db):
                pltpu.sync_copy(rb, acc.at[db], add=True)

            load_idx(ebase, isrc0, idst0)
            gather_start(isrc0, rows0, sem0)

            def pair(p, _):
                j0 = ebase + 2 * p * CS
                load_idx(j0 + CS, isrc1, idst1)
                gather_start(isrc1, rows1, sem1)
                gather_wait(isrc0, rows0, sem0)
                scatter(rows0, idst0)

                @pl.when(p < NPAIR - 1)
                def _():
                    load_idx(j0 + 2 * CS, isrc0, idst0)
                    gather_start(isrc0, rows0, sem0)

                gather_wait(isrc1, rows1, sem1)
                scatter(rows1, idst1)
                return 0

            lax.fori_loop(0, NPAIR, pair, 0)
            plsc.subcore_barrier()

            def outcopy(off, sz):
                pltpu.sync_copy(acc.at[pl.ds(off, sz)], rows0.at[pl.ds(0, sz)])
                pltpu.sync_copy(rows0.at[pl.ds(0, sz)],
                                outs[oi].at[pl.ds(off, sz), pl.ds(16 * g, 16)])

            stripe_io(outcopy)
            plsc.subcore_barrier()


def _sc_scatter(tA, tB, sadjs, dsts):
    return pl.kernel(
        _sc_scatter_body,
        compiler_params=pltpu.CompilerParams(use_tc_tiling_on_sc=False),
        out_type=tuple(jax.ShapeDtypeStruct((N, 128), jnp.float32) for _ in range(3)),
        mesh=_MESH,
        scratch_types=[
            pltpu.VMEM_SHARED((N, QUART), jnp.float32),
            pltpu.VMEM((CS,), jnp.int32),
            pltpu.VMEM((CS,), jnp.int32),
            pltpu.VMEM((CS,), jnp.int32),
            pltpu.VMEM((CS,), jnp.int32),
            pltpu.VMEM((CS, QUART), jnp.float32),
            pltpu.VMEM((CS, QUART), jnp.float32),
            pltpu.SemaphoreType.DMA,
            pltpu.SemaphoreType.DMA,
        ],
    )(tA, tB, *sadjs, *dsts)


R_POST = 2000


def _tc_post_kernel(sa_ref, sb_ref, sc_ref, xa_ref, xb_ref, cnt_ref, w_ref,
                    b_ref, oa_ref, ob_ref):
    inv = 1.0 / jnp.maximum(cnt_ref[...], 1.0)
    lane = lax.broadcasted_iota(jnp.int32, (R_POST, 128), 1)
    low = lane < 64

    def scaled(ref, el, eh):
        pat = jnp.where(low, inv[:, el:el + 1], inv[:, eh:eh + 1])
        return ref[...] * pat

    sa = scaled(sa_ref, 1, 2)
    sb = scaled(sb_ref, 3, 0)
    sc = jnp.where(low, sc_ref[...] * inv[:, 4:5], 0.0)

    def mm(x, k):
        return jnp.dot(x, w_ref[k], preferred_element_type=jnp.float32)

    z_ch = mm(sa, 0) + mm(sb, 1)
    z_gw = mm(sb, 2)
    z_hru = mm(sc, 3)
    r_gw = mm(xb_ref[...], 4)
    r_ch = mm(xa_ref[...], 5)
    r_hru = mm(xa_ref[...], 6)
    hru = jnp.maximum(z_hru + r_hru + b_ref[0:1, :], 0.0)
    ch = jnp.maximum((z_ch + r_ch + b_ref[1:2, :]) / 3.0, 0.0)
    gw = jnp.maximum(z_gw + r_gw + b_ref[2:3, :], 0.0)
    oa_ref[...] = jnp.concatenate([hru, ch], axis=1)
    ob_ref[...] = jnp.concatenate([gw, hru], axis=1)


def _tc_post(SA, SB, SC2, XA, XB, cntT, w, bsum):
    grid = (N // R_POST,)
    blk = pl.BlockSpec((R_POST, 128), lambda i: (i, 0))
    return pl.pallas_call(
        _tc_post_kernel,
        grid=grid,
        in_specs=[
            blk, blk, blk, blk, blk,
            pl.BlockSpec((R_POST, NLAYER_TYPES), lambda i: (i, 0)),
            pl.BlockSpec((7, 128, D), lambda i: (0, 0, 0)),
            pl.BlockSpec((3, D), lambda i: (0, 0)),
        ],
        out_specs=[blk, blk],
        out_shape=[jax.ShapeDtypeStruct((N, 128), jnp.float32),
                   jax.ShapeDtypeStruct((N, 128), jnp.float32)],
    )(SA, SB, SC2, XA, XB, cntT, w, bsum)


R_POOL = 2000


def _tc_pool_kernel(xa_ref, b_ref, w1p_ref, w1t_ref, td_ref, b1_ref, w2_ref,
                    o_ref, accp, accc):
    i = pl.program_id(0)

    @pl.when(i == 0)
    def _():
        accp[...] = jnp.zeros_like(accp)
        accc[...] = jnp.zeros_like(accc)

    ids = b_ref[0]
    iota = lax.broadcasted_iota(jnp.int32, (B, R_POOL), 0)
    m = (iota == ids).astype(jnp.float32)
    accp[...] += jnp.dot(m, xa_ref[...], preferred_element_type=jnp.float32)
    accc[...] = accc[...] + jnp.sum(m, axis=1, keepdims=True)

    @pl.when(i == (N // R_POOL) - 1)
    def _():
        pooled = accp[...] / jnp.maximum(accc[...], 1.0)
        h = jnp.maximum(jnp.dot(pooled, w1p_ref[...], preferred_element_type=jnp.float32)
                        + jnp.dot(td_ref[...], w1t_ref[...], preferred_element_type=jnp.float32)
                        + b1_ref[...], 0.0)
        o_ref[...] = h * w2_ref[...]


def _tc_pool_head(XA, batch3, w1p, w1t, train_data, b1b, w2):
    grid = (N // R_POOL,)
    return pl.pallas_call(
        _tc_pool_kernel,
        grid=grid,
        in_specs=[
            pl.BlockSpec((R_POOL, 128), lambda i: (i, 0)),
            pl.BlockSpec((1, 1, R_POOL), lambda i: (i, 0, 0)),
            pl.BlockSpec((128, 128), lambda i: (0, 0)),
            pl.BlockSpec((TD, 128), lambda i: (0, 0)),
            pl.BlockSpec((B, TD), lambda i: (0, 0)),
            pl.BlockSpec((B, 128), lambda i: (0, 0)),
            pl.BlockSpec((1, 128), lambda i: (0, 0)),
        ],
        out_specs=pl.BlockSpec((B, 128), lambda i: (0, 0)),
        out_shape=jax.ShapeDtypeStruct((B, 128), jnp.float32),
        scratch_shapes=[
            pltpu.VMEM((B, 128), jnp.float32),
            pltpu.VMEM((B, 128), jnp.float32),
        ],
    )(XA, batch3, w1p, w1t, train_data, b1b, w2)


def kernel(x_hru, x_channel, x_gw_cell, ei_sw_gw, ei_hydro, ei_sw, ei_gw_sw,
           ei_self, batch, train_data, Wl, bl, Wr, fc1_w, fc1_b, fc2_w, fc2_b):
    eis = [ei_sw_gw, ei_hydro, ei_sw, ei_gw_sw, ei_self]
    srcs = [ei[0] for ei in eis]
    dsts = [ei[1] for ei in eis]

    XA = jnp.concatenate([x_hru, x_channel], axis=1)
    XB = jnp.concatenate([x_gw_cell, x_hru], axis=1)

    sadjs = [
        jnp.concatenate([8 * srcs[e] + (GBASE[e] + qc) for qc in range(4)])
        for e in range(NLAYER_TYPES)
    ]

    cnts = _sc_counts(*dsts)
    cntT = jnp.stack(cnts, axis=1)

    z64 = jnp.zeros((64, 64), jnp.float32)

    def wstack(l):
        wr_ch = (Wr[l, 1] + Wr[l, 2] + Wr[l, 3]).T
        return jnp.stack([
            jnp.concatenate([Wl[l, 1].T, Wl[l, 2].T], axis=0),
            jnp.concatenate([Wl[l, 3].T, z64], axis=0),
            jnp.concatenate([z64, Wl[l, 0].T], axis=0),
            jnp.concatenate([Wl[l, 4].T, z64], axis=0),
            jnp.concatenate([Wr[l, 0].T, z64], axis=0),
            jnp.concatenate([z64, wr_ch], axis=0),
            jnp.concatenate([Wr[l, 4].T, z64], axis=0),
        ])
    w = jnp.stack([wstack(l) for l in range(L)])
    bsum = jnp.stack([bl[:, 4], bl[:, 1] + bl[:, 2] + bl[:, 3], bl[:, 0]],
                     axis=1)

    for l in range(L):
        SA, SB, SC2 = _sc_scatter(XA.reshape(8 * N, QUART), XB.reshape(8 * N, QUART),
                                  sadjs, dsts)
        XA, XB = _tc_post(SA, SB, SC2, XA, XB, cntT, w[l], bsum[l])

    batch3 = batch.reshape(N // R_POOL, 1, R_POOL)
    w1p = jnp.concatenate([jnp.zeros((64, 128), jnp.float32), fc1_w.T[:D]], axis=0)
    w1t = fc1_w.T[D:]
    hw = _tc_pool_head(XA, batch3, w1p, w1t, train_data,
                       jnp.broadcast_to(fc1_b, (B, 128)), fc2_w)
    return jnp.sum(hw, axis=1, keepdims=True) + fc2_b


# speedup vs baseline: 12.5112x; 1.1273x over previous
import functools

import jax
import jax.numpy as jnp
from jax import lax
from jax.experimental import pallas as pl
from jax.experimental.pallas import tpu as pltpu
from jax.experimental.pallas import tpu_sc as plsc

N = 50000
E = 800000
D = 64
HALF = 32
B = 16
TD = 16
L = 2
NLAYER_TYPES = 5
SRC_TYPE = (0, 1, 0, 2, 0)
C_EDGE = 2000
N_CHUNK = (E // 16) // C_EDGE
QUART = 16
CS = 1000
NPAIR = (E // 16) // CS // 2
STRIPE_A = 3128
SUBCHUNKS_A = ((0, 1000), (1000, 1000), (2000, 1000), (3000, 128))
SUBCHUNKS_B = ((0, 1000), (1000, 1000), (2000, 1000), (3000, 80))

_MESH = plsc.VectorSubcoreMesh(core_axis_name="c", subcore_axis_name="s")


def _sc_counts_body(d0, d1, d2, d3, d4, o0, o1, o2, o3, o4, acc, idst, ones, zb, bounce):
    c = lax.axis_index("c")
    s = lax.axis_index("s")
    dsts = [d0, d1, d2, d3, d4]
    outs = [o0, o1, o2, o3, o4]

    def fill(i, _):
        ones[pl.ds(i * 16, 16)] = jnp.ones((16,), jnp.float32)
        zb[pl.ds(i * 16, 16)] = jnp.zeros((16,), jnp.float32)
        return 0

    lax.fori_loop(0, 3200 // 16, fill, 0)

    for e in range(NLAYER_TYPES):
        @pl.when(c == e % 2)
        def _():
            @pl.when(s < 15)
            def _():
                pltpu.sync_copy(zb, acc.at[pl.ds(s * 3200, 3200)])

            @pl.when(s == 15)
            def _():
                pltpu.sync_copy(zb.at[pl.ds(0, 2000)], acc.at[pl.ds(48000, 2000)])

            plsc.subcore_barrier()

            def chunk(j, _):
                off = s * (E // 16) + j * C_EDGE
                pltpu.sync_copy(dsts[e].at[pl.ds(off, C_EDGE)], idst)
                pltpu.sync_copy(ones.at[pl.ds(0, C_EDGE)], acc.at[idst], add=True)
                return 0

            lax.fori_loop(0, N_CHUNK, chunk, 0)
            plsc.subcore_barrier()

            @pl.when(s < 15)
            def _():
                pltpu.sync_copy(acc.at[pl.ds(s * 3200, 3200)], bounce)
                pltpu.sync_copy(bounce, outs[e].at[pl.ds(s * 3200, 3200)])

            @pl.when(s == 15)
            def _():
                pltpu.sync_copy(acc.at[pl.ds(48000, 2000)], bounce.at[pl.ds(0, 2000)])
                pltpu.sync_copy(bounce.at[pl.ds(0, 2000)], outs[e].at[pl.ds(48000, 2000)])

            plsc.subcore_barrier()


def _sc_counts(d0, d1, d2, d3, d4):
    return pl.kernel(
        _sc_counts_body,
        compiler_params=pltpu.CompilerParams(use_tc_tiling_on_sc=False),
        out_type=tuple(jax.ShapeDtypeStruct((N,), jnp.float32) for _ in range(NLAYER_TYPES)),
        mesh=_MESH,
        scratch_types=[
            pltpu.VMEM_SHARED((N,), jnp.float32),
            pltpu.VMEM((C_EDGE,), jnp.int32),
            pltpu.VMEM((3200,), jnp.float32),
            pltpu.VMEM((3200,), jnp.float32),
            pltpu.VMEM((3200,), jnp.float32),
        ],
    )(d0, d1, d2, d3, d4)


TSEL = (0, 0, 0, 1, 0)
GBASE = (0, 4, 0, 0, 0)
SMAP = ((1, 4), (0, 0), (0, 4), (1, 0), (2, 0))


def _sc_scatter_body(tA, tB, s0, s1, s2, s3, s4, d0, d1, d2, d3, d4,
                     oA, oB, oC, acc,
                     isrcA, isrcB, isrcC, idstA, idstB, idstC,
                     rowsA, rowsB, rowsC,
                     sgA, sgB, sgC, ssA, ssB, ssC):
    c = lax.axis_index("c")
    s = lax.axis_index("s")
    tables = [tA, tB]
    sadjs = [s0, s1, s2, s3, s4]
    dsts = [d0, d1, d2, d3, d4]
    outs = [oA, oB, oC]
    isrc = [isrcA, isrcB, isrcC]
    idst = [idstA, idstB, idstC]
    rows = [rowsA, rowsB, rowsC]
    sg = [sgA, sgB, sgC]
    ss = [ssA, ssB, ssC]
    ebase = s * (E // 16)
    NCH = (E // 16) // CS
    NTRIP = (NCH - 2) // 3

    def stripe_io(body):
        @pl.when(s < 15)
        def _():
            for off, sz in SUBCHUNKS_A:
                body(s * STRIPE_A + off, sz)

        @pl.when(s == 15)
        def _():
            for off, sz in SUBCHUNKS_B:
                body(15 * STRIPE_A + off, sz)

    for e in range(NLAYER_TYPES):
        table = tables[TSEL[e]]
        oi, goff = SMAP[e]
        for q in range(2):
            qbase = (2 * c + q) * E
            g = goff + 2 * c + q

            def fillz(i, _):
                rowsA[i, pl.ds(0, 16)] = jnp.zeros((16,), jnp.float32)
                return 0

            lax.fori_loop(0, CS, fillz, 0)

            def zero(off, sz):
                pltpu.sync_copy(rowsA.at[pl.ds(0, sz)], acc.at[pl.ds(off, sz)])

            stripe_io(zero)
            plsc.subcore_barrier()

            def load_idx(j, b):
                off = ebase + j * CS
                pltpu.sync_copy(sadjs[e].at[pl.ds(qbase + off, CS)], isrc[b])
                pltpu.sync_copy(dsts[e].at[pl.ds(off, CS)], idst[b])

            def gather_start(b):
                pltpu.async_copy(table.at[isrc[b]], rows[b], sg[b])

            def gather_wait(b):
                pltpu.make_async_copy(table.at[isrc[b]], rows[b], sg[b]).wait()

            def scatter_start(b):
                pltpu.async_copy(rows[b], acc.at[idst[b]], ss[b], add=True)

            def scatter_wait(b):
                pltpu.make_async_copy(rows[b], acc.at[idst[b]], ss[b]).wait()

            load_idx(0, 0)
            gather_start(0)

            def slot(p, b, guard):
                nb = (b + 1) % 3
                if guard:
                    @pl.when(p > 0)
                    def _():
                        scatter_wait(nb)
                else:
                    scatter_wait(nb)
                load_idx(3 * p + b + 1, nb)
                gather_start(nb)
                gather_wait(b)
                scatter_start(b)

            def triple(p, _):
                slot(p, 0, True)
                slot(p, 1, True)
                slot(p, 2, False)
                return 0

            lax.fori_loop(0, NTRIP, triple, 0)
            for j in range(3 * NTRIP, NCH - 1):
                b = j % 3
                nb = (j + 1) % 3
                scatter_wait(nb)
                load_idx(j + 1, nb)
                gather_start(nb)
                gather_wait(b)
                scatter_start(b)
            bl_ = (NCH - 1) % 3
            gather_wait(bl_)
            scatter_start(bl_)
            scatter_wait((NCH - 3) % 3)
            scatter_wait((NCH - 2) % 3)
            scatter_wait(bl_)
            plsc.subcore_barrier()

            def outcopy(off, sz):
                pltpu.sync_copy(acc.at[pl.ds(off, sz)], rowsA.at[pl.ds(0, sz)])
                pltpu.sync_copy(rowsA.at[pl.ds(0, sz)],
                                outs[oi].at[pl.ds(off, sz), pl.ds(16 * g, 16)])

            stripe_io(outcopy)
            plsc.subcore_barrier()


def _sc_scatter(tA, tB, sadjs, dsts):
    return pl.kernel(
        _sc_scatter_body,
        compiler_params=pltpu.CompilerParams(use_tc_tiling_on_sc=False),
        out_type=tuple(jax.ShapeDtypeStruct((N, 128), jnp.float32) for _ in range(3)),
        mesh=_MESH,
        scratch_types=[
            pltpu.VMEM_SHARED((N, QUART), jnp.float32),
            pltpu.VMEM((CS,), jnp.int32),
            pltpu.VMEM((CS,), jnp.int32),
            pltpu.VMEM((CS,), jnp.int32),
            pltpu.VMEM((CS,), jnp.int32),
            pltpu.VMEM((CS,), jnp.int32),
            pltpu.VMEM((CS,), jnp.int32),
            pltpu.VMEM((CS, QUART), jnp.float32),
            pltpu.VMEM((CS, QUART), jnp.float32),
            pltpu.VMEM((CS, QUART), jnp.float32),
            pltpu.SemaphoreType.DMA,
            pltpu.SemaphoreType.DMA,
            pltpu.SemaphoreType.DMA,
            pltpu.SemaphoreType.DMA,
            pltpu.SemaphoreType.DMA,
            pltpu.SemaphoreType.DMA,
        ],
    )(tA, tB, *sadjs, *dsts)


R_POST = 2000


def _tc_post_kernel(sa_ref, sb_ref, sc_ref, xa_ref, xb_ref, cnt_ref, w_ref,
                    b_ref, oa_ref, ob_ref):
    inv = 1.0 / jnp.maximum(cnt_ref[...], 1.0)
    lane = lax.broadcasted_iota(jnp.int32, (R_POST, 128), 1)
    low = lane < 64

    def scaled(ref, el, eh):
        pat = jnp.where(low, inv[:, el:el + 1], inv[:, eh:eh + 1])
        return ref[...] * pat

    sa = scaled(sa_ref, 1, 2)
    sb = scaled(sb_ref, 3, 0)
    sc = jnp.where(low, sc_ref[...] * inv[:, 4:5], 0.0)

    def mm(x, k):
        return jnp.dot(x, w_ref[k], preferred_element_type=jnp.float32)

    z_ch = mm(sa, 0) + mm(sb, 1)
    z_gw = mm(sb, 2)
    z_hru = mm(sc, 3)
    r_gw = mm(xb_ref[...], 4)
    r_ch = mm(xa_ref[...], 5)
    r_hru = mm(xa_ref[...], 6)
    hru = jnp.maximum(z_hru + r_hru + b_ref[0:1, :], 0.0)
    ch = jnp.maximum((z_ch + r_ch + b_ref[1:2, :]) / 3.0, 0.0)
    gw = jnp.maximum(z_gw + r_gw + b_ref[2:3, :], 0.0)
    oa_ref[...] = jnp.concatenate([hru, ch], axis=1)
    ob_ref[...] = jnp.concatenate([gw, hru], axis=1)


def _tc_post(SA, SB, SC2, XA, XB, cntT, w, bsum):
    grid = (N // R_POST,)
    blk = pl.BlockSpec((R_POST, 128), lambda i: (i, 0))
    return pl.pallas_call(
        _tc_post_kernel,
        grid=grid,
        in_specs=[
            blk, blk, blk, blk, blk,
            pl.BlockSpec((R_POST, NLAYER_TYPES), lambda i: (i, 0)),
            pl.BlockSpec((7, 128, D), lambda i: (0, 0, 0)),
            pl.BlockSpec((3, D), lambda i: (0, 0)),
        ],
        out_specs=[blk, blk],
        out_shape=[jax.ShapeDtypeStruct((N, 128), jnp.float32),
                   jax.ShapeDtypeStruct((N, 128), jnp.float32)],
    )(SA, SB, SC2, XA, XB, cntT, w, bsum)


R_POOL = 2000


def _tc_pool_kernel(xa_ref, b_ref, w1p_ref, w1t_ref, td_ref, b1_ref, w2_ref,
                    o_ref, accp, accc):
    i = pl.program_id(0)

    @pl.when(i == 0)
    def _():
        accp[...] = jnp.zeros_like(accp)
        accc[...] = jnp.zeros_like(accc)

    ids = b_ref[0]
    iota = lax.broadcasted_iota(jnp.int32, (B, R_POOL), 0)
    m = (iota == ids).astype(jnp.float32)
    accp[...] += jnp.dot(m, xa_ref[...], preferred_element_type=jnp.float32)
    accc[...] = accc[...] + jnp.sum(m, axis=1, keepdims=True)

    @pl.when(i == (N // R_POOL) - 1)
    def _():
        pooled = accp[...] / jnp.maximum(accc[...], 1.0)
        h = jnp.maximum(jnp.dot(pooled, w1p_ref[...], preferred_element_type=jnp.float32)
                        + jnp.dot(td_ref[...], w1t_ref[...], preferred_element_type=jnp.float32)
                        + b1_ref[...], 0.0)
        o_ref[...] = h * w2_ref[...]


def _tc_pool_head(XA, batch3, w1p, w1t, train_data, b1b, w2):
    grid = (N // R_POOL,)
    return pl.pallas_call(
        _tc_pool_kernel,
        grid=grid,
        in_specs=[
            pl.BlockSpec((R_POOL, 128), lambda i: (i, 0)),
            pl.BlockSpec((1, 1, R_POOL), lambda i: (i, 0, 0)),
            pl.BlockSpec((128, 128), lambda i: (0, 0)),
            pl.BlockSpec((TD, 128), lambda i: (0, 0)),
            pl.BlockSpec((B, TD), lambda i: (0, 0)),
            pl.BlockSpec((B, 128), lambda i: (0, 0)),
            pl.BlockSpec((1, 128), lambda i: (0, 0)),
        ],
        out_specs=pl.BlockSpec((B, 128), lambda i: (0, 0)),
        out_shape=jax.ShapeDtypeStruct((B, 128), jnp.float32),
        scratch_shapes=[
            pltpu.VMEM((B, 128), jnp.float32),
            pltpu.VMEM((B, 128), jnp.float32),
        ],
    )(XA, batch3, w1p, w1t, train_data, b1b, w2)


def kernel(x_hru, x_channel, x_gw_cell, ei_sw_gw, ei_hydro, ei_sw, ei_gw_sw,
           ei_self, batch, train_data, Wl, bl, Wr, fc1_w, fc1_b, fc2_w, fc2_b):
    eis = [ei_sw_gw, ei_hydro, ei_sw, ei_gw_sw, ei_self]
    srcs = [ei[0] for ei in eis]
    dsts = [ei[1] for ei in eis]

    XA = jnp.concatenate([x_hru, x_channel], axis=1)
    XB = jnp.concatenate([x_gw_cell, x_hru], axis=1)

    sadjs = [
        jnp.concatenate([8 * srcs[e] + (GBASE[e] + qc) for qc in range(4)])
        for e in range(NLAYER_TYPES)
    ]

    cnts = _sc_counts(*dsts)
    cntT = jnp.stack(cnts, axis=1)

    z64 = jnp.zeros((64, 64), jnp.float32)

    def wstack(l):
        wr_ch = (Wr[l, 1] + Wr[l, 2] + Wr[l, 3]).T
        return jnp.stack([
            jnp.concatenate([Wl[l, 1].T, Wl[l, 2].T], axis=0),
            jnp.concatenate([Wl[l, 3].T, z64], axis=0),
            jnp.concatenate([z64, Wl[l, 0].T], axis=0),
            jnp.concatenate([Wl[l, 4].T, z64], axis=0),
            jnp.concatenate([Wr[l, 0].T, z64], axis=0),
            jnp.concatenate([z64, wr_ch], axis=0),
            jnp.concatenate([Wr[l, 4].T, z64], axis=0),
        ])
    w = jnp.stack([wstack(l) for l in range(L)])
    bsum = jnp.stack([bl[:, 4], bl[:, 1] + bl[:, 2] + bl[:, 3], bl[:, 0]],
                     axis=1)

    for l in range(L):
        SA, SB, SC2 = _sc_scatter(XA.reshape(8 * N, QUART), XB.reshape(8 * N, QUART),
                                  sadjs, dsts)
        XA, XB = _tc_post(SA, SB, SC2, XA, XB, cntT, w[l], bsum[l])

    batch3 = batch.reshape(N // R_POOL, 1, R_POOL)
    w1p = jnp.concatenate([jnp.zeros((64, 128), jnp.float32), fc1_w.T[:D]], axis=0)
    w1t = fc1_w.T[D:]
    hw = _tc_pool_head(XA, batch3, w1p, w1t, train_data,
                       jnp.broadcast_to(fc1_b, (B, 128)), fc2_w)
    return jnp.sum(hw, axis=1, keepdims=True) + fc2_b
